# baseline (speedup 1.0000x reference)
import numpy as np

import jax
import jax.numpy as jnp
from jax import lax
from jax.experimental import pallas as pl
from jax.experimental.pallas import tpu as pltpu
from jax.experimental.pallas import tpu_sc as plsc

_RAD = 0.2
_N = 10000
_NPAD = 10240
_ROWS = 8
_COLS = 1280
_S = 5000
_K = 64
_SPAD = 5120
_CPT = 160
_RAD2_F = np.float32(_RAD * _RAD)
_RAD2_BITS = int(_RAD2_F.view(np.int32))


def _fps_body(px_ref, py_ref, pz_ref, pxs_ref, pys_ref, pzs_ref,
              pos_out_ref, dist_ref):
    idx2d = (lax.broadcasted_iota(jnp.int32, (_ROWS, _COLS), 0) * _COLS
             + lax.broadcasted_iota(jnp.int32, (_ROWS, _COLS), 1))
    dist_ref[...] = jnp.where(idx2d < _N, jnp.inf, -jnp.inf).astype(jnp.float32)
    lane = lax.broadcasted_iota(jnp.int32, (1, 128), 1)

    def make_row(lx, ly, lz):
        return jnp.where(lane == 0, lx,
                         jnp.where(lane == 1, ly,
                                   jnp.where(lane == 2, lz, jnp.float32(0.0))))

    def body(i, carry):
        lx, ly, lz = carry
        pxv = px_ref[...]
        pyv = py_ref[...]
        pzv = pz_ref[...]
        dx = pxv - lx
        dy = pyv - ly
        dz = pzv - lz
        d = dx * dx + dy * dy + dz * dz
        dist = jnp.minimum(dist_ref[...], d)
        dist_ref[...] = dist
        pos_out_ref[pl.ds(i - 1, 1), :] = make_row(lx, ly, lz)
        m = jnp.max(dist)
        ii = (lax.broadcasted_iota(jnp.int32, (_ROWS, _COLS), 0) * _COLS
              + lax.broadcasted_iota(jnp.int32, (_ROWS, _COLS), 1))
        nxt = jnp.min(jnp.where(dist == m, ii, jnp.int32(2**30)))
        nlx = pxs_ref[nxt]
        nly = pys_ref[nxt]
        nlz = pzs_ref[nxt]
        return (nlx, nly, nlz)

    lx, ly, lz = lax.fori_loop(
        1, _S, body, (px_ref[0, 0], py_ref[0, 0], pz_ref[0, 0]))
    pos_out_ref[pl.ds(_S - 1, 1), :] = make_row(lx, ly, lz)


def _fps_pos_rows(point):
    p = point.astype(jnp.float32)
    pad = jnp.zeros((_NPAD - _N,), jnp.float32)
    px = jnp.concatenate([p[:, 0], pad]).reshape(_ROWS, _COLS)
    py = jnp.concatenate([p[:, 1], pad]).reshape(_ROWS, _COLS)
    pz = jnp.concatenate([p[:, 2], pad]).reshape(_ROWS, _COLS)
    pxf = px.reshape(-1)
    pyf = py.reshape(-1)
    pzf = pz.reshape(-1)
    return pl.pallas_call(
        _fps_body,
        in_specs=[
            pl.BlockSpec(memory_space=pltpu.VMEM),
            pl.BlockSpec(memory_space=pltpu.VMEM),
            pl.BlockSpec(memory_space=pltpu.VMEM),
            pl.BlockSpec(memory_space=pltpu.SMEM),
            pl.BlockSpec(memory_space=pltpu.SMEM),
            pl.BlockSpec(memory_space=pltpu.SMEM),
        ],
        out_shape=jax.ShapeDtypeStruct((_S, 128), jnp.float32),
        scratch_shapes=[pltpu.VMEM((_ROWS, _COLS), jnp.float32)],
    )(px, py, pz, pxf, pyf, pzf)


def _u_body(xp_ref, w_ref, b_ref, u_ref):
    acc = lax.dot_general(xp_ref[...], w_ref[...],
                          (((1,), (0,)), ((), ())),
                          preferred_element_type=jnp.float32)
    u_ref[...] = acc + b_ref[...]


def _u_table(xyz, point, W1, b1):
    xp = jnp.concatenate([xyz, point], axis=1)
    xp = jnp.pad(xp, ((0, _NPAD - _N), (0, 5)))
    w = jnp.pad(W1, ((0, 5), (0, 0)))
    return pl.pallas_call(
        _u_body,
        grid=(8,),
        in_specs=[
            pl.BlockSpec((1280, 136), lambda i: (i, 0)),
            pl.BlockSpec((136, 128), lambda i: (0, 0)),
            pl.BlockSpec((1, 128), lambda i: (0, 0)),
        ],
        out_specs=pl.BlockSpec((1280, 128), lambda i: (i, 0)),
        out_shape=jax.ShapeDtypeStruct((_NPAD, 128), jnp.float32),
    )(xp, w, b1.reshape(1, 128))


_K3_B = 160


def _thresh_body(pos_ref, pt_ref, psn_ref, pn_ref, tsel_ref, kc_ref, d2_ref):
    dot = lax.dot_general(pos_ref[...], pt_ref[...],
                          (((1,), (0,)), ((), ())),
                          preferred_element_type=jnp.float32)
    d2 = (psn_ref[...] + pn_ref[...]) - 2.0 * dot
    d2_ref[...] = d2
    rad2 = jnp.float32(_RAD2_F)
    cv = jnp.sum(jnp.where(d2 <= rad2, 1.0, 0.0), axis=1, keepdims=True)
    kcf = jnp.minimum(cv, jnp.float32(_K))

    def bs_body(_, lohi):
        lo, hi = lohi
        mid = (lo + hi) // 2
        midf = lax.bitcast_convert_type(mid, jnp.float32)
        cnt = jnp.sum(jnp.where(d2_ref[...] <= midf, 1.0, 0.0),
                      axis=1, keepdims=True)
        ge = cnt >= kcf
        return (jnp.where(ge, lo, mid + 1), jnp.where(ge, mid, hi))

    lo0 = jnp.zeros((_K3_B, 1), jnp.int32)
    hi0 = jnp.full((_K3_B, 1), _RAD2_BITS, jnp.int32)
    _, t64 = lax.fori_loop(0, 31, bs_body, (lo0, hi0))
    t64f = lax.bitcast_convert_type(t64, jnp.float32)
    d2v = d2_ref[...]
    dn = jnp.min(jnp.where(d2v > t64f, d2v, jnp.float32(1e30)),
                 axis=1, keepdims=True)
    dnb = lax.bitcast_convert_type(dn, jnp.int32)
    tsel_ref[...] = lax.bitcast_convert_type((t64 + dnb) // 2, jnp.float32)
    kc_ref[...] = kcf.astype(jnp.int32)


def _thresholds(pos_rows, point, pos_s):
    ppad = jnp.pad(point.astype(jnp.float32), ((0, _NPAD - _N), (0, 0)))
    pt = jnp.pad(ppad.T, ((0, 125), (0, 0)))
    pn = jnp.concatenate([jnp.sum(point ** 2, axis=1),
                          jnp.full((_NPAD - _N,), 1e30, jnp.float32)])
    psn = jnp.pad(jnp.sum(pos_s ** 2, axis=1)[:, None],
                  ((0, _SPAD - _S), (0, 0)))
    pos_pad = jnp.pad(pos_rows, ((0, _SPAD - _S), (0, 0)))
    return pl.pallas_call(
        _thresh_body,
        grid=(_SPAD // _K3_B,),
        in_specs=[
            pl.BlockSpec((_K3_B, 128), lambda i: (i, 0)),
            pl.BlockSpec((128, _NPAD), lambda i: (0, 0)),
            pl.BlockSpec((_K3_B, 1), lambda i: (i, 0)),
            pl.BlockSpec((1, _NPAD), lambda i: (0, 0)),
        ],
        out_specs=[
            pl.BlockSpec((_K3_B, 1), lambda i: (i, 0)),
            pl.BlockSpec((_K3_B, 1), lambda i: (i, 0)),
            pl.BlockSpec((_K3_B, _NPAD), lambda i: (i, 0)),
        ],
        out_shape=[
            jax.ShapeDtypeStruct((_SPAD, 1), jnp.float32),
            jax.ShapeDtypeStruct((_SPAD, 1), jnp.int32),
            jax.ShapeDtypeStruct((_SPAD, _NPAD), jnp.float32),
        ],
    )(pos_pad, pt, psn, pn.reshape(1, _NPAD))


def _sc_body(d2h, tsh, u_hbm, g_hbm, rowv, tsv, bigbuf, colbuf, gbuf,
             sem_g, sem_row, sem_out):
    wid = lax.axis_index("s") * 2 + lax.axis_index("c")
    base_c = wid * _CPT
    pltpu.sync_copy(tsh.at[pl.ds(base_c * 16, _CPT * 16)], tsv)

    zero16 = jnp.zeros((16,), jnp.int32)
    lanes = lax.iota(jnp.int32, 16)
    lane64 = lanes * _K

    pltpu.async_copy(d2h.at[base_c], rowv.at[pl.ds(0, _NPAD)], sem_row)
    for s in range(4):
        colbuf[pl.ds(_K + s * 16, 16)] = zero16
    pltpu.async_copy(u_hbm.at[colbuf.at[pl.ds(_K, _K)]],
                     gbuf.at[pl.ds(_K, _K)], sem_g)
    pltpu.async_copy(gbuf.at[pl.ds(0, _K)],
                     g_hbm.at[pl.ds(base_c * _K, _K)], sem_out)

    def per_centroid(ci, carry):
        p = lax.rem(ci, 2)
        pn = lax.rem(ci + 1, 2)
        t = tsv[pl.ds(ci * 16, 16)]
        nxtrow = jnp.minimum(ci + 1, _CPT - 1)
        pltpu.async_copy(d2h.at[base_c + nxtrow],
                         rowv.at[pl.ds(pn * _NPAD, _NPAD)], sem_row)
        pltpu.make_async_copy(d2h.at[base_c],
                              rowv.at[pl.ds(p * _NPAD, _NPAD)],
                              sem_row).wait()
        pbase = p * _NPAD
        cbase = p * _K

        def chunk(c, percnt):
            b = c * 16
            d2c = rowv[pl.ds(pbase + b, 16)]
            mskc = jnp.logical_and(d2c <= t, percnt < _K)
            plsc.store_scatter(bigbuf, [lane64 + percnt], lanes + b,
                               mask=mskc)
            return percnt + jnp.where(mskc, jnp.int32(1), jnp.int32(0))

        cnt = lax.fori_loop(0, _NPAD // 16, chunk,
                            jnp.zeros((16,), jnp.int32), unroll=4)
        colbuf[pl.ds(cbase, 16)] = zero16
        colbuf[pl.ds(cbase + 16, 16)] = zero16
        colbuf[pl.ds(cbase + 32, 16)] = zero16
        colbuf[pl.ds(cbase + 48, 16)] = zero16
        base = jnp.int32(0)
        for l in range(16):
            cl = cnt[l]
            for s in range(4):
                seg = bigbuf[pl.ds(l * _K + s * 16, 16)]
                posm = cbase + base + (s * 16 + lanes)
                mm = jnp.logical_and(s * 16 + lanes < cl,
                                     base + s * 16 + lanes < _K)
                plsc.store_scatter(colbuf, [posm], seg, mask=mm)
            base = base + cl
        gprev = gbuf.at[pl.ds(pn * _K, _K)]
        pltpu.make_async_copy(u_hbm.at[colbuf.at[pl.ds(cbase, _K)]],
                              gprev, sem_g).wait()
        prow = jnp.maximum(ci - 1, 0)
        pltpu.async_copy(gprev, g_hbm.at[pl.ds((base_c + prow) * _K, _K)],
                         sem_out)
        gcur = gbuf.at[pl.ds(p * _K, _K)]
        pltpu.make_async_copy(gcur, g_hbm.at[pl.ds(base_c * _K, _K)],
                              sem_out).wait()
        pltpu.async_copy(u_hbm.at[colbuf.at[pl.ds(cbase, _K)]], gcur,
                         sem_g)
        return carry

    lax.fori_loop(0, _CPT, per_centroid, 0)
    glast = gbuf.at[pl.ds(((_CPT - 1) % 2) * _K, _K)]
    pltpu.make_async_copy(u_hbm.at[colbuf.at[pl.ds(0, _K)]], glast,
                          sem_g).wait()
    pltpu.async_copy(glast, g_hbm.at[pl.ds((base_c + _CPT - 1) * _K, _K)],
                     sem_out)
    pltpu.make_async_copy(d2h.at[base_c], rowv.at[pl.ds(0, _NPAD)],
                          sem_row).wait()
    pltpu.make_async_copy(gbuf.at[pl.ds(0, _K)],
                          g_hbm.at[pl.ds(base_c * _K, _K)], sem_out).wait()
    pltpu.make_async_copy(gbuf.at[pl.ds(0, _K)],
                          g_hbm.at[pl.ds(base_c * _K, _K)], sem_out).wait()


def _sc_select_gather(d2, tsel, u):
    tsrep = jnp.broadcast_to(tsel, (_SPAD, 16)).reshape(-1)
    mesh = plsc.VectorSubcoreMesh(core_axis_name="c", subcore_axis_name="s")
    fn = pl.kernel(
        _sc_body,
        mesh=mesh,
        compiler_params=pltpu.CompilerParams(needs_layout_passes=False),
        out_type=jax.ShapeDtypeStruct((_SPAD * _K, 128), jnp.float32),
        scratch_types=[
            pltpu.VMEM((2 * _NPAD,), jnp.float32),
            pltpu.VMEM((_CPT * 16,), jnp.float32),
            pltpu.VMEM((16 * _K,), jnp.int32),
            pltpu.VMEM((2 * _K,), jnp.int32),
            pltpu.VMEM((2 * _K, 128), jnp.float32),
            pltpu.SemaphoreType.DMA,
            pltpu.SemaphoreType.DMA,
            pltpu.SemaphoreType.DMA,
        ],
    )
    return fn(d2, tsrep, u)


_K5_B = 40


def _mlp_body(g_ref, pos_ref, kc_ref, w1b_ref, w2_ref, b2_ref, out_ref):
    v = lax.dot_general(pos_ref[...], w1b_ref[...],
                        (((1,), (0,)), ((), ())),
                        preferred_element_type=jnp.float32)
    v_exp = jnp.broadcast_to(v[:, None, :], (_K5_B, _K, 128)).reshape(
        _K5_B * _K, 128)
    a = jnp.maximum(g_ref[...] - v_exp, 0.0)
    h = lax.dot_general(a, w2_ref[...], (((1,), (0,)), ((), ())),
                        preferred_element_type=jnp.float32)
    slot = lax.broadcasted_iota(jnp.int32, (_K5_B * _K, 1), 0) % _K
    kc_exp = jnp.broadcast_to(kc_ref[...][:, None, :],
                              (_K5_B, _K, 1)).reshape(_K5_B * _K, 1)
    hm = jnp.where(slot < kc_exp, h, -jnp.inf)
    mx = jnp.max(hm.reshape(_K5_B, _K, 128), axis=1)
    y = mx + b2_ref[...]
    out_ref[...] = jnp.where(jnp.isfinite(y), y, 0.0)


def _edge_mlp(g, pos_rows, kc, W1, W2, b2):
    w1b = jnp.pad(W1[128:131], ((0, 125), (0, 0)))
    return pl.pallas_call(
        _mlp_body,
        grid=(_S // _K5_B,),
        in_specs=[
            pl.BlockSpec((_K5_B * _K, 128), lambda i: (i, 0)),
            pl.BlockSpec((_K5_B, 128), lambda i: (i, 0)),
            pl.BlockSpec((_K5_B, 1), lambda i: (i, 0)),
            pl.BlockSpec((128, 128), lambda i: (0, 0)),
            pl.BlockSpec((128, 128), lambda i: (0, 0)),
            pl.BlockSpec((1, 128), lambda i: (0, 0)),
        ],
        out_specs=pl.BlockSpec((_K5_B, 128), lambda i: (i, 0)),
        out_shape=jax.ShapeDtypeStruct((_S, 128), jnp.float32),
    )(g, pos_rows, kc, w1b, W2, b2.reshape(1, 128))


def kernel(xyz, point, batch, num_samples, W1, b1, W2, b2):
    pos_rows = _fps_pos_rows(point)
    pos_s = pos_rows[:, :3]
    u = _u_table(xyz, point, W1, b1)
    tsel, kc, d2 = _thresholds(pos_rows, point, pos_s)
    g = _sc_select_gather(d2, tsel, u)
    out = _edge_mlp(g, pos_rows, kc[:_S], W1, W2, b2)
    batch_s = jnp.zeros((_S,), batch.dtype)
    return (out, pos_s, batch_s)


# speedup vs baseline: 1.0093x; 1.0093x over previous
import numpy as np

import jax
import jax.numpy as jnp
from jax import lax
from jax.experimental import pallas as pl
from jax.experimental.pallas import tpu as pltpu
from jax.experimental.pallas import tpu_sc as plsc

_RAD = 0.2
_N = 10000
_NPAD = 10240
_ROWS = 8
_COLS = 1280
_S = 5000
_K = 64
_SPAD = 5120
_CPT = 160
_RAD2_F = np.float32(_RAD * _RAD)
_RAD2_BITS = int(_RAD2_F.view(np.int32))


def _fps_body(px_ref, py_ref, pz_ref, pxs_ref, pys_ref, pzs_ref,
              pos_out_ref, dist_ref):
    idx2d = (lax.broadcasted_iota(jnp.int32, (_ROWS, _COLS), 0) * _COLS
             + lax.broadcasted_iota(jnp.int32, (_ROWS, _COLS), 1))
    dist_ref[...] = jnp.where(idx2d < _N, jnp.inf, -jnp.inf).astype(jnp.float32)
    lane = lax.broadcasted_iota(jnp.int32, (1, 128), 1)

    def make_row(lx, ly, lz):
        return jnp.where(lane == 0, lx,
                         jnp.where(lane == 1, ly,
                                   jnp.where(lane == 2, lz, jnp.float32(0.0))))

    def body(i, carry):
        lx, ly, lz = carry
        pxv = px_ref[...]
        pyv = py_ref[...]
        pzv = pz_ref[...]
        dx = pxv - lx
        dy = pyv - ly
        dz = pzv - lz
        d = dx * dx + dy * dy + dz * dz
        dist = jnp.minimum(dist_ref[...], d)
        dist_ref[...] = dist
        pos_out_ref[pl.ds(i - 1, 1), :] = make_row(lx, ly, lz)
        m = jnp.max(dist)
        ii = (lax.broadcasted_iota(jnp.int32, (_ROWS, _COLS), 0) * _COLS
              + lax.broadcasted_iota(jnp.int32, (_ROWS, _COLS), 1))
        nxt = jnp.min(jnp.where(dist == m, ii, jnp.int32(2**30)))
        nlx = pxs_ref[nxt]
        nly = pys_ref[nxt]
        nlz = pzs_ref[nxt]
        return (nlx, nly, nlz)

    lx, ly, lz = lax.fori_loop(
        1, _S, body, (px_ref[0, 0], py_ref[0, 0], pz_ref[0, 0]),
        unroll=2)
    pos_out_ref[pl.ds(_S - 1, 1), :] = make_row(lx, ly, lz)


def _fps_pos_rows(point):
    p = point.astype(jnp.float32)
    pad = jnp.zeros((_NPAD - _N,), jnp.float32)
    px = jnp.concatenate([p[:, 0], pad]).reshape(_ROWS, _COLS)
    py = jnp.concatenate([p[:, 1], pad]).reshape(_ROWS, _COLS)
    pz = jnp.concatenate([p[:, 2], pad]).reshape(_ROWS, _COLS)
    pxf = px.reshape(-1)
    pyf = py.reshape(-1)
    pzf = pz.reshape(-1)
    return pl.pallas_call(
        _fps_body,
        in_specs=[
            pl.BlockSpec(memory_space=pltpu.VMEM),
            pl.BlockSpec(memory_space=pltpu.VMEM),
            pl.BlockSpec(memory_space=pltpu.VMEM),
            pl.BlockSpec(memory_space=pltpu.SMEM),
            pl.BlockSpec(memory_space=pltpu.SMEM),
            pl.BlockSpec(memory_space=pltpu.SMEM),
        ],
        out_shape=jax.ShapeDtypeStruct((_S, 128), jnp.float32),
        scratch_shapes=[pltpu.VMEM((_ROWS, _COLS), jnp.float32)],
    )(px, py, pz, pxf, pyf, pzf)


def _u_body(xp_ref, w_ref, b_ref, u_ref):
    acc = lax.dot_general(xp_ref[...], w_ref[...],
                          (((1,), (0,)), ((), ())),
                          preferred_element_type=jnp.float32)
    u_ref[...] = acc + b_ref[...]


def _u_table(xyz, point, W1, b1):
    xp = jnp.concatenate([xyz, point], axis=1)
    xp = jnp.pad(xp, ((0, _NPAD - _N), (0, 5)))
    w = jnp.pad(W1, ((0, 5), (0, 0)))
    return pl.pallas_call(
        _u_body,
        grid=(8,),
        in_specs=[
            pl.BlockSpec((1280, 136), lambda i: (i, 0)),
            pl.BlockSpec((136, 128), lambda i: (0, 0)),
            pl.BlockSpec((1, 128), lambda i: (0, 0)),
        ],
        out_specs=pl.BlockSpec((1280, 128), lambda i: (i, 0)),
        out_shape=jax.ShapeDtypeStruct((_NPAD, 128), jnp.float32),
    )(xp, w, b1.reshape(1, 128))


_K3_B = 160


def _thresh_body(pos_ref, pt_ref, psn_ref, pn_ref, tsel_ref, kc_ref, d2_ref):
    dot = lax.dot_general(pos_ref[...], pt_ref[...],
                          (((1,), (0,)), ((), ())),
                          preferred_element_type=jnp.float32)
    d2 = (psn_ref[...] + pn_ref[...]) - 2.0 * dot
    d2_ref[...] = d2
    rad2 = jnp.float32(_RAD2_F)
    cv = jnp.sum(jnp.where(d2 <= rad2, 1.0, 0.0), axis=1, keepdims=True)
    kcf = jnp.minimum(cv, jnp.float32(_K))

    def bs_body(_, lohi):
        lo, hi = lohi
        mid = (lo + hi) // 2
        midf = lax.bitcast_convert_type(mid, jnp.float32)
        cnt = jnp.sum(jnp.where(d2_ref[...] <= midf, 1.0, 0.0),
                      axis=1, keepdims=True)
        ge = cnt >= kcf
        return (jnp.where(ge, lo, mid + 1), jnp.where(ge, mid, hi))

    lo0 = jnp.zeros((_K3_B, 1), jnp.int32)
    hi0 = jnp.full((_K3_B, 1), _RAD2_BITS, jnp.int32)
    _, t64 = lax.fori_loop(0, 31, bs_body, (lo0, hi0))
    t64f = lax.bitcast_convert_type(t64, jnp.float32)
    d2v = d2_ref[...]
    dn = jnp.min(jnp.where(d2v > t64f, d2v, jnp.float32(1e30)),
                 axis=1, keepdims=True)
    dnb = lax.bitcast_convert_type(dn, jnp.int32)
    tsel_ref[...] = lax.bitcast_convert_type((t64 + dnb) // 2, jnp.float32)
    kc_ref[...] = kcf.astype(jnp.int32)


def _thresholds(pos_rows, point, pos_s):
    ppad = jnp.pad(point.astype(jnp.float32), ((0, _NPAD - _N), (0, 0)))
    pt = jnp.pad(ppad.T, ((0, 125), (0, 0)))
    pn = jnp.concatenate([jnp.sum(point ** 2, axis=1),
                          jnp.full((_NPAD - _N,), 1e30, jnp.float32)])
    psn = jnp.pad(jnp.sum(pos_s ** 2, axis=1)[:, None],
                  ((0, _SPAD - _S), (0, 0)))
    pos_pad = jnp.pad(pos_rows, ((0, _SPAD - _S), (0, 0)))
    return pl.pallas_call(
        _thresh_body,
        grid=(_SPAD // _K3_B,),
        in_specs=[
            pl.BlockSpec((_K3_B, 128), lambda i: (i, 0)),
            pl.BlockSpec((128, _NPAD), lambda i: (0, 0)),
            pl.BlockSpec((_K3_B, 1), lambda i: (i, 0)),
            pl.BlockSpec((1, _NPAD), lambda i: (0, 0)),
        ],
        out_specs=[
            pl.BlockSpec((_K3_B, 1), lambda i: (i, 0)),
            pl.BlockSpec((_K3_B, 1), lambda i: (i, 0)),
            pl.BlockSpec((_K3_B, _NPAD), lambda i: (i, 0)),
        ],
        out_shape=[
            jax.ShapeDtypeStruct((_SPAD, 1), jnp.float32),
            jax.ShapeDtypeStruct((_SPAD, 1), jnp.int32),
            jax.ShapeDtypeStruct((_SPAD, _NPAD), jnp.float32),
        ],
    )(pos_pad, pt, psn, pn.reshape(1, _NPAD))


def _sc_body(d2h, tsh, u_hbm, g_hbm, rowv, tsv, bigbuf, colbuf, gbuf,
             sem_g, sem_row, sem_out):
    wid = lax.axis_index("s") * 2 + lax.axis_index("c")
    base_c = wid * _CPT
    pltpu.sync_copy(tsh.at[pl.ds(base_c * 16, _CPT * 16)], tsv)

    zero16 = jnp.zeros((16,), jnp.int32)
    lanes = lax.iota(jnp.int32, 16)
    lane64 = lanes * _K

    pltpu.async_copy(d2h.at[base_c], rowv.at[pl.ds(0, _NPAD)], sem_row)
    for s in range(4):
        colbuf[pl.ds(_K + s * 16, 16)] = zero16
    pltpu.async_copy(u_hbm.at[colbuf.at[pl.ds(_K, _K)]],
                     gbuf.at[pl.ds(_K, _K)], sem_g)
    pltpu.async_copy(gbuf.at[pl.ds(0, _K)],
                     g_hbm.at[pl.ds(base_c * _K, _K)], sem_out)

    def per_centroid(ci, carry):
        p = lax.rem(ci, 2)
        pn = lax.rem(ci + 1, 2)
        t = tsv[pl.ds(ci * 16, 16)]
        nxtrow = jnp.minimum(ci + 1, _CPT - 1)
        pltpu.async_copy(d2h.at[base_c + nxtrow],
                         rowv.at[pl.ds(pn * _NPAD, _NPAD)], sem_row)
        pltpu.make_async_copy(d2h.at[base_c],
                              rowv.at[pl.ds(p * _NPAD, _NPAD)],
                              sem_row).wait()
        pbase = p * _NPAD
        cbase = p * _K

        def chunk(c, percnt):
            b = c * 16
            d2c = rowv[pl.ds(pbase + b, 16)]
            mskc = jnp.logical_and(d2c <= t, percnt < _K)
            plsc.store_scatter(bigbuf, [lane64 + percnt], lanes + b,
                               mask=mskc)
            return percnt + jnp.where(mskc, jnp.int32(1), jnp.int32(0))

        cnt = lax.fori_loop(0, _NPAD // 16, chunk,
                            jnp.zeros((16,), jnp.int32), unroll=4)
        colbuf[pl.ds(cbase, 16)] = zero16
        colbuf[pl.ds(cbase + 16, 16)] = zero16
        colbuf[pl.ds(cbase + 32, 16)] = zero16
        colbuf[pl.ds(cbase + 48, 16)] = zero16
        base = jnp.int32(0)
        for l in range(16):
            cl = cnt[l]
            for s in range(4):
                seg = bigbuf[pl.ds(l * _K + s * 16, 16)]
                posm = cbase + base + (s * 16 + lanes)
                mm = jnp.logical_and(s * 16 + lanes < cl,
                                     base + s * 16 + lanes < _K)
                plsc.store_scatter(colbuf, [posm], seg, mask=mm)
            base = base + cl
        gprev = gbuf.at[pl.ds(pn * _K, _K)]
        pltpu.make_async_copy(u_hbm.at[colbuf.at[pl.ds(cbase, _K)]],
                              gprev, sem_g).wait()
        prow = jnp.maximum(ci - 1, 0)
        pltpu.async_copy(gprev, g_hbm.at[pl.ds((base_c + prow) * _K, _K)],
                         sem_out)
        gcur = gbuf.at[pl.ds(p * _K, _K)]
        pltpu.make_async_copy(gcur, g_hbm.at[pl.ds(base_c * _K, _K)],
                              sem_out).wait()
        pltpu.async_copy(u_hbm.at[colbuf.at[pl.ds(cbase, _K)]], gcur,
                         sem_g)
        return carry

    lax.fori_loop(0, _CPT, per_centroid, 0)
    glast = gbuf.at[pl.ds(((_CPT - 1) % 2) * _K, _K)]
    pltpu.make_async_copy(u_hbm.at[colbuf.at[pl.ds(0, _K)]], glast,
                          sem_g).wait()
    pltpu.async_copy(glast, g_hbm.at[pl.ds((base_c + _CPT - 1) * _K, _K)],
                     sem_out)
    pltpu.make_async_copy(d2h.at[base_c], rowv.at[pl.ds(0, _NPAD)],
                          sem_row).wait()
    pltpu.make_async_copy(gbuf.at[pl.ds(0, _K)],
                          g_hbm.at[pl.ds(base_c * _K, _K)], sem_out).wait()
    pltpu.make_async_copy(gbuf.at[pl.ds(0, _K)],
                          g_hbm.at[pl.ds(base_c * _K, _K)], sem_out).wait()


def _sc_select_gather(d2, tsel, u):
    tsrep = jnp.broadcast_to(tsel, (_SPAD, 16)).reshape(-1)
    mesh = plsc.VectorSubcoreMesh(core_axis_name="c", subcore_axis_name="s")
    fn = pl.kernel(
        _sc_body,
        mesh=mesh,
        compiler_params=pltpu.CompilerParams(needs_layout_passes=False),
        out_type=jax.ShapeDtypeStruct((_SPAD * _K, 128), jnp.float32),
        scratch_types=[
            pltpu.VMEM((2 * _NPAD,), jnp.float32),
            pltpu.VMEM((_CPT * 16,), jnp.float32),
            pltpu.VMEM((16 * _K,), jnp.int32),
            pltpu.VMEM((2 * _K,), jnp.int32),
            pltpu.VMEM((2 * _K, 128), jnp.float32),
            pltpu.SemaphoreType.DMA,
            pltpu.SemaphoreType.DMA,
            pltpu.SemaphoreType.DMA,
        ],
    )
    return fn(d2, tsrep, u)


_K5_B = 40


def _mlp_body(g_ref, pos_ref, kc_ref, w1b_ref, w2_ref, b2_ref, out_ref):
    v = lax.dot_general(pos_ref[...], w1b_ref[...],
                        (((1,), (0,)), ((), ())),
                        preferred_element_type=jnp.float32)
    v_exp = jnp.broadcast_to(v[:, None, :], (_K5_B, _K, 128)).reshape(
        _K5_B * _K, 128)
    a = jnp.maximum(g_ref[...] - v_exp, 0.0)
    h = lax.dot_general(a, w2_ref[...], (((1,), (0,)), ((), ())),
                        preferred_element_type=jnp.float32)
    slot = lax.broadcasted_iota(jnp.int32, (_K5_B * _K, 1), 0) % _K
    kc_exp = jnp.broadcast_to(kc_ref[...][:, None, :],
                              (_K5_B, _K, 1)).reshape(_K5_B * _K, 1)
    hm = jnp.where(slot < kc_exp, h, -jnp.inf)
    mx = jnp.max(hm.reshape(_K5_B, _K, 128), axis=1)
    y = mx + b2_ref[...]
    out_ref[...] = jnp.where(jnp.isfinite(y), y, 0.0)


def _edge_mlp(g, pos_rows, kc, W1, W2, b2):
    w1b = jnp.pad(W1[128:131], ((0, 125), (0, 0)))
    return pl.pallas_call(
        _mlp_body,
        grid=(_S // _K5_B,),
        in_specs=[
            pl.BlockSpec((_K5_B * _K, 128), lambda i: (i, 0)),
            pl.BlockSpec((_K5_B, 128), lambda i: (i, 0)),
            pl.BlockSpec((_K5_B, 1), lambda i: (i, 0)),
            pl.BlockSpec((128, 128), lambda i: (0, 0)),
            pl.BlockSpec((128, 128), lambda i: (0, 0)),
            pl.BlockSpec((1, 128), lambda i: (0, 0)),
        ],
        out_specs=pl.BlockSpec((_K5_B, 128), lambda i: (i, 0)),
        out_shape=jax.ShapeDtypeStruct((_S, 128), jnp.float32),
    )(g, pos_rows, kc, w1b, W2, b2.reshape(1, 128))


def kernel(xyz, point, batch, num_samples, W1, b1, W2, b2):
    pos_rows = _fps_pos_rows(point)
    pos_s = pos_rows[:, :3]
    u = _u_table(xyz, point, W1, b1)
    tsel, kc, d2 = _thresholds(pos_rows, point, pos_s)
    g = _sc_select_gather(d2, tsel, u)
    out = _edge_mlp(g, pos_rows, kc[:_S], W1, W2, b2)
    batch_s = jnp.zeros((_S,), batch.dtype)
    return (out, pos_s, batch_s)
